# baseline (speedup 1.0000x reference)
import functools

import jax
import jax.numpy as jnp
from jax import lax
from jax.experimental import pallas as pl
from jax.experimental.pallas import tpu as pltpu
from jax.experimental.pallas import tpu_sc as plsc

_W = 384


def kernel(x, emb_table, pos_embd):
    B, S = x.shape
    V, D = emb_table.shape
    info = plsc.get_sparse_core_info()
    NC, NS, L = info.num_cores, info.num_subcores, info.num_lanes
    NW = NC * NS
    EPW = B // NW

    x = x.astype(jnp.int32)
    pad = jnp.zeros((_W,), jnp.int32)
    xh = jnp.concatenate([(x >> 1).reshape(B * S), pad])
    xp = jnp.concatenate([(x & 1).reshape(B * S), pad])
    tbl2 = emb_table.reshape(V // 2, 2 * D)

    mesh = plsc.VectorSubcoreMesh(core_axis_name="c", subcore_axis_name="s")

    @functools.partial(
        pl.kernel,
        mesh=mesh,
        compiler_params=pltpu.CompilerParams(use_tc_tiling_on_sc=True),
        out_type=jax.ShapeDtypeStruct((B, S, D), jnp.float32),
        scratch_types=[
            pltpu.VMEM((_W,), jnp.int32),
            pltpu.VMEM((_W,), jnp.int32),
            pltpu.VMEM((_W,), jnp.int32),
            pltpu.VMEM((_W,), jnp.int32),
            pltpu.VMEM((2, S, 2 * D), jnp.float32),
            pltpu.VMEM((2, S, D), jnp.float32),
            pltpu.VMEM((S * D,), jnp.float32),
        ]
        + [pltpu.SemaphoreType.DMA] * 6,
    )
    def emb_kernel(
        xh_hbm,
        xp_hbm,
        tbl_hbm,
        pos_hbm,
        out_hbm,
        xhw0,
        xhw1,
        xpw0,
        xpw1,
        g_v,
        o_v,
        pos_v,
        *sems,
    ):
        xhw = (xhw0, xhw1)
        xpw = (xpw0, xpw1)
        gsem = sems[0:2]
        ssem = sems[2:4]
        isem = sems[4:6]
        wid = lax.axis_index("s") * NC + lax.axis_index("c")
        e0 = wid * EPW
        r0 = e0 * S

        pltpu.sync_copy(pos_hbm, pos_v)

        def astart(c):
            return ((r0 + c * S) // 128) * 128

        def woff(c):
            return (r0 + c * S) - astart(c)

        def start_idx(c, b):
            a = astart(c)
            pltpu.async_copy(xh_hbm.at[pl.ds(a, _W)], xhw[b], isem[b])
            pltpu.async_copy(xp_hbm.at[pl.ds(a, _W)], xpw[b], isem[b])

        def wait_idx(c, b):
            a = astart(c)
            pltpu.make_async_copy(xh_hbm.at[pl.ds(a, _W)], xhw[b], isem[b]).wait()
            pltpu.make_async_copy(xp_hbm.at[pl.ds(a, _W)], xpw[b], isem[b]).wait()

        def start_gather(c, b):
            pltpu.async_copy(
                tbl_hbm.at[xhw[b].at[pl.ds(woff(c), S)]], g_v.at[b], gsem[b]
            )

        def wait_gather(c, b):
            pltpu.make_async_copy(
                tbl_hbm.at[xhw[b].at[pl.ds(woff(c), S)]], g_v.at[b], gsem[b]
            ).wait()

        def start_store(c, b):
            pltpu.async_copy(o_v.at[b], out_hbm.at[e0 + c], ssem[b])

        def wait_store(c, b):
            pltpu.make_async_copy(o_v.at[b], out_hbm.at[e0 + c], ssem[b]).wait()

        start_idx(0, 0)
        start_idx(1, 1)
        wait_idx(0, 0)
        start_gather(0, 0)

        @pl.loop(0, EPW, step=2)
        def ring(g):
            for k in range(2):
                c = g + k
                b = k

                @pl.when(c + 1 < EPW)
                def _():
                    wait_idx(c + 1, 1 - b)
                    start_gather(c + 1, 1 - b)

                wait_gather(c, b)

                @pl.when(c >= 2)
                def _():
                    wait_store(c - 2, b)

                wo = woff(c)

                def add_group(base_r):
                    pv = xpw[b][pl.ds(wo + base_r, L)]
                    for j in range(L):
                        r = base_r + j
                        po = pl.multiple_of(pv[j] * D, D)
                        for d in range(D // L):
                            o_v[b, r, pl.ds(d * L, L)] = g_v[
                                b, r, pl.ds(po + d * L, L)
                            ] + pos_v[pl.ds(r * D + d * L, L)]

                @pl.loop(0, S // L)
                def row_grp(t):
                    add_group(t * L)

                if S % L:
                    add_group(S - L)

                start_store(c, b)

                @pl.when(c + 2 < EPW)
                def _():
                    start_idx(c + 2, b)

        wait_store(EPW - 2, (EPW - 2) % 2)
        wait_store(EPW - 1, (EPW - 1) % 2)

    return emb_kernel(xh, xp, tbl2, pos_embd.reshape(S * D))


# speedup vs baseline: 1.2345x; 1.2345x over previous
import functools

import jax
import jax.numpy as jnp
from jax import lax
from jax.experimental import pallas as pl
from jax.experimental.pallas import tpu as pltpu
from jax.experimental.pallas import tpu_sc as plsc

_NBUF = 4


def kernel(x, emb_table, pos_embd):
    B, S = x.shape
    V, D = emb_table.shape
    info = plsc.get_sparse_core_info()
    NC, NS, L = info.num_cores, info.num_subcores, info.num_lanes
    NW = NC * NS
    EPW = B // NW

    mesh = plsc.VectorSubcoreMesh(core_axis_name="c", subcore_axis_name="s")

    @functools.partial(
        pl.kernel,
        mesh=mesh,
        compiler_params=pltpu.CompilerParams(use_tc_tiling_on_sc=False),
        out_type=jax.ShapeDtypeStruct((B, S, D), jnp.float32),
        scratch_types=[
            pltpu.VMEM((EPW, S), jnp.int32),
            pltpu.VMEM((_NBUF, S, D), jnp.float32),
            pltpu.VMEM((S, D), jnp.float32),
        ]
        + [pltpu.SemaphoreType.DMA] * (2 * _NBUF),
    )
    def emb_kernel(x_hbm, table_hbm, pos_hbm, out_hbm, idx_all, rows_v, pos_v, *sems):
        gsem = sems[:_NBUF]
        ssem = sems[_NBUF:]
        wid = lax.axis_index("s") * NC + lax.axis_index("c")
        e0 = wid * EPW

        pltpu.sync_copy(pos_hbm, pos_v)
        pltpu.sync_copy(x_hbm.at[pl.ds(e0, EPW)], idx_all)

        def start_gather(c, b):
            pltpu.async_copy(table_hbm.at[idx_all.at[c]], rows_v.at[b], gsem[b])

        def wait_gather(c, b):
            pltpu.make_async_copy(
                table_hbm.at[idx_all.at[c]], rows_v.at[b], gsem[b]
            ).wait()

        def start_store(c, b):
            pltpu.async_copy(rows_v.at[b], out_hbm.at[e0 + c], ssem[b])

        def wait_store(c, b):
            pltpu.make_async_copy(rows_v.at[b], out_hbm.at[e0 + c], ssem[b]).wait()

        start_gather(0, 0)
        start_gather(1, 1)

        @pl.loop(0, EPW, step=_NBUF)
        def ring(g):
            for k in range(_NBUF):
                c = g + k
                b = k

                wait_gather(c, b)

                @pl.loop(0, S)
                def row_add(r):
                    for d in range(D // L):
                        sl = pl.ds(d * L, L)
                        plsc.addupdate(rows_v.at[b, r, sl], pos_v[r, sl])

                start_store(c, b)

                nb = (k + 2) % _NBUF

                @pl.when(c >= 2)
                def _():
                    wait_store(c + 2 - _NBUF, nb)

                @pl.when(c + 2 < EPW)
                def _():
                    start_gather(c + 2, nb)

        wait_store(EPW - 2, (EPW - 2) % _NBUF)
        wait_store(EPW - 1, (EPW - 1) % _NBUF)

    return emb_kernel(x.astype(jnp.int32), emb_table, pos_embd)


# speedup vs baseline: 1.3962x; 1.1309x over previous
import functools

import jax
import jax.numpy as jnp
from jax import lax
from jax.experimental import pallas as pl
from jax.experimental.pallas import tpu as pltpu
from jax.experimental.pallas import tpu_sc as plsc


def kernel(x, emb_table, pos_embd):
    B, S = x.shape
    V, D = emb_table.shape
    info = plsc.get_sparse_core_info()
    NC, NS, L = info.num_cores, info.num_subcores, info.num_lanes
    NW = NC * NS
    EPW = B // NW

    mesh = plsc.VectorSubcoreMesh(core_axis_name="c", subcore_axis_name="s")

    @functools.partial(
        pl.kernel,
        mesh=mesh,
        compiler_params=pltpu.CompilerParams(use_tc_tiling_on_sc=False),
        out_type=jax.ShapeDtypeStruct((B, S * D), jnp.float32),
        scratch_types=[
            pltpu.VMEM((EPW, S), jnp.int32),
            pltpu.VMEM((2, S, D), jnp.float32),
            pltpu.VMEM((2, S * D), jnp.float32),
            pltpu.VMEM((S, D), jnp.float32),
        ]
        + [pltpu.SemaphoreType.DMA] * 4,
    )
    def emb_kernel(x_hbm, table_hbm, pos_hbm, out_hbm, idx_all, g_v, o_v, pos_v, *sems):
        gsem = sems[:2]
        ssem = sems[2:]
        wid = lax.axis_index("s") * NC + lax.axis_index("c")
        e0 = wid * EPW

        pltpu.sync_copy(pos_hbm, pos_v)
        pltpu.sync_copy(x_hbm.at[pl.ds(e0, EPW)], idx_all)

        def start_gather(c, b):
            pltpu.async_copy(table_hbm.at[idx_all.at[c]], g_v.at[b], gsem[b])

        def wait_gather(c, b):
            pltpu.make_async_copy(table_hbm.at[idx_all.at[c]], g_v.at[b], gsem[b]).wait()

        def start_store(c, b):
            pltpu.async_copy(o_v.at[b], out_hbm.at[e0 + c], ssem[b])

        def wait_store(c, b):
            pltpu.make_async_copy(o_v.at[b], out_hbm.at[e0 + c], ssem[b]).wait()

        start_gather(0, 0)

        @pl.loop(0, EPW, step=2)
        def ring(g):
            for k in range(2):
                c = g + k
                b = k

                @pl.when(c + 1 < EPW)
                def _():
                    start_gather(c + 1, 1 - b)

                wait_gather(c, b)

                @pl.when(c >= 2)
                def _():
                    wait_store(c - 2, b)

                @pl.loop(0, S)
                def row_add(r):
                    for d in range(D // L):
                        sl = pl.ds(d * L, L)
                        o_v[b, pl.ds(r * D + d * L, L)] = g_v[b, r, sl] + pos_v[r, sl]

                start_store(c, b)

        wait_store(EPW - 2, EPW % 2)
        wait_store(EPW - 1, 1 - EPW % 2)

    return emb_kernel(x.astype(jnp.int32), emb_table, pos_embd).reshape(B, S, D)
